# baseline (speedup 1.0000x reference)
import jax
import jax.numpy as jnp
from jax.experimental import pallas as pl
from jax.experimental.pallas import tpu as pltpu


_NC = 8
_DEPTH = 3


def _dist_kernel(x_ref, w_hbm, out_hbm, wbuf, obuf, lsem, ssem):
    N = w_hbm.shape[0]
    CH = N // _NC

    def load(i):
        slot = i % _DEPTH
        return pltpu.make_async_copy(
            w_hbm.at[pl.ds(i * CH, CH), :], wbuf.at[slot], lsem.at[slot]
        )

    def store(i):
        slot = i % _DEPTH
        return pltpu.make_async_copy(
            obuf.at[slot], out_hbm.at[:, pl.ds(i * CH, CH)], ssem.at[slot]
        )

    for j in range(_DEPTH):
        load(j).start()

    x = x_ref[:]
    x2 = jnp.sum(x * x, axis=1, keepdims=True)

    for i in range(_NC):
        slot = i % _DEPTH
        load(i).wait()
        if i >= _DEPTH:
            store(i - _DEPTH).wait()
        w = wbuf[slot]
        xw = jax.lax.dot_general(
            x, w,
            dimension_numbers=(((1,), (1,)), ((), ())),
            preferred_element_type=jnp.float32,
        )
        w2 = jnp.sum(w * w, axis=1, keepdims=True).T
        obuf[slot] = (x2 + w2) - 2.0 * xw
        store(i).start()
        if i + _DEPTH < _NC:
            load(i + _DEPTH).start()

    for j in range(_DEPTH):
        store(_NC - _DEPTH + j).wait()


def kernel(x, weights):
    B, D = x.shape
    R, C, D2 = weights.shape
    N = R * C
    CH = N // _NC
    w = weights.reshape(N, D2)
    out = pl.pallas_call(
        _dist_kernel,
        in_specs=[
            pl.BlockSpec(memory_space=pltpu.MemorySpace.VMEM),
            pl.BlockSpec(memory_space=pltpu.MemorySpace.HBM),
        ],
        out_specs=pl.BlockSpec(memory_space=pltpu.MemorySpace.HBM),
        out_shape=jax.ShapeDtypeStruct((B, N), jnp.float32),
        scratch_shapes=[
            pltpu.VMEM((_DEPTH, CH, D2), jnp.float32),
            pltpu.VMEM((_DEPTH, B, CH), jnp.float32),
            pltpu.SemaphoreType.DMA((_DEPTH,)),
            pltpu.SemaphoreType.DMA((_DEPTH,)),
        ],
    )(x, w)
    return out.reshape(B, R, C)


# speedup vs baseline: 1.0234x; 1.0234x over previous
import jax
import jax.numpy as jnp
from jax.experimental import pallas as pl
from jax.experimental.pallas import tpu as pltpu


_NC = 4


def _dist_kernel(x_ref, w_hbm, out_hbm, wbuf, obuf, lsem, ssem):
    N = w_hbm.shape[0]
    CH = N // _NC
    H = N // 2

    def load(i):
        slot = i % 2
        return pltpu.make_async_copy(
            w_hbm.at[pl.ds(i * CH, CH), :], wbuf.at[slot], lsem.at[slot]
        )

    def store(h):
        return pltpu.make_async_copy(
            obuf.at[:, pl.ds(h * H, H)], out_hbm.at[:, pl.ds(h * H, H)],
            ssem.at[h],
        )

    load(0).start()
    load(1).start()

    x = x_ref[:]
    x2 = jnp.sum(x * x, axis=1, keepdims=True)

    for i in range(_NC):
        slot = i % 2
        load(i).wait()
        w = wbuf[slot]
        xw = jax.lax.dot_general(
            x, w,
            dimension_numbers=(((1,), (1,)), ((), ())),
            preferred_element_type=jnp.float32,
        )
        w2 = jnp.sum(w * w, axis=1, keepdims=True).T
        obuf[:, pl.ds(i * CH, CH)] = (x2 + w2) - 2.0 * xw
        if i + 2 < _NC:
            load(i + 2).start()
        if (i + 1) * CH == H:
            store(0).start()

    store(1).start()
    store(0).wait()
    store(1).wait()


def kernel(x, weights):
    B, D = x.shape
    R, C, D2 = weights.shape
    N = R * C
    CH = N // _NC
    w = weights.reshape(N, D2)
    out = pl.pallas_call(
        _dist_kernel,
        in_specs=[
            pl.BlockSpec(memory_space=pltpu.MemorySpace.VMEM),
            pl.BlockSpec(memory_space=pltpu.MemorySpace.HBM),
        ],
        out_specs=pl.BlockSpec(memory_space=pltpu.MemorySpace.HBM),
        out_shape=jax.ShapeDtypeStruct((B, N), jnp.float32),
        scratch_shapes=[
            pltpu.VMEM((2, CH, D2), jnp.float32),
            pltpu.VMEM((B, N), jnp.float32),
            pltpu.SemaphoreType.DMA((2,)),
            pltpu.SemaphoreType.DMA((2,)),
        ],
    )(x, w)
    return out.reshape(B, R, C)


# speedup vs baseline: 1.1001x; 1.0749x over previous
import jax
import jax.numpy as jnp
from jax.experimental import pallas as pl


def _dist_kernel(x_ref, w_ref, out_ref):
    x = x_ref[:]
    w = w_ref[:]
    xw = jax.lax.dot_general(
        x, w,
        dimension_numbers=(((1,), (1,)), ((), ())),
        preferred_element_type=jnp.float32,
        precision=jax.lax.Precision.DEFAULT,
    )
    x2 = jnp.sum(x * x, axis=1, keepdims=True)
    w2 = jnp.sum(w * w, axis=1, keepdims=True).T
    out_ref[:] = (x2 + w2) - 2.0 * xw


def kernel(x, weights):
    B, D = x.shape
    R, C, D2 = weights.shape
    N = R * C
    w = weights.reshape(N, D2)
    NBLK = 2
    NB = N // NBLK
    out = pl.pallas_call(
        _dist_kernel,
        grid=(NBLK,),
        in_specs=[
            pl.BlockSpec((B, D), lambda i: (0, 0)),
            pl.BlockSpec((NB, D2), lambda i: (i, 0)),
        ],
        out_specs=pl.BlockSpec((B, NB), lambda i: (0, i)),
        out_shape=jax.ShapeDtypeStruct((B, N), jnp.float32),
    )(x, w)
    return out.reshape(B, R, C)
